# baseline (speedup 1.0000x reference)
import functools

import jax
import jax.numpy as jnp
from jax import lax
from jax.experimental import pallas as pl
from jax.experimental.pallas import tpu as pltpu
from jax.experimental.pallas import tpu_sc as plsc

_VOCAB = 1000000
_D = 64
_OUT = 100
_B = 16384
_H = 50

_NC, _NS = 2, 16
_NW = _NC * _NS
_BAGS_W = _B // _NW
_PAIR = 2
_ROWS_CHUNK = _PAIR * _H
_NCHUNK = _BAGS_W // _PAIR
_NBUF = 2
_LG = _D // 16
_FLUSH = 32

_CB = 2176
_GRID = (_VOCAB // 2 + _CB - 1) // _CB
_VPACK = _GRID * _CB


def _tc_pairpack(embT):
    def body(in_a, in_b, eye_ref, out_ref):
        ta = jax.lax.dot_general(
            in_a[...], eye_ref[...],
            dimension_numbers=(((0,), (0,)), ((), ())),
            preferred_element_type=jnp.float32)
        tb = jax.lax.dot_general(
            in_b[...], eye_ref[...],
            dimension_numbers=(((0,), (0,)), ((), ())),
            preferred_element_type=jnp.float32)
        out_ref[...] = jnp.concatenate([ta, tb], axis=1)

    return pl.pallas_call(
        body,
        grid=(_GRID,),
        in_specs=[
            pl.BlockSpec((_D, _CB), lambda i: (0, 2 * i)),
            pl.BlockSpec((_D, _CB), lambda i: (0, 2 * i + 1)),
            pl.BlockSpec((_D, _D), lambda i: (0, 0)),
        ],
        out_specs=pl.BlockSpec((_CB, 2 * _D), lambda i: (i, 0)),
        out_shape=jax.ShapeDtypeStruct((_VPACK, 2 * _D), jnp.float32),
    )(embT, embT, jnp.eye(_D, dtype=jnp.float32))


def _sc_bag_sum(xr, emb2):
    mesh = plsc.VectorSubcoreMesh(
        core_axis_name="c", subcore_axis_name="s",
        num_cores=_NC, num_subcores=_NS)

    @functools.partial(
        pl.kernel,
        out_type=jax.ShapeDtypeStruct((_B, _D), jnp.float32),
        mesh=mesh,
        compiler_params=pltpu.CompilerParams(
            use_tc_tiling_on_sc=True, needs_layout_passes=False),
        scratch_types=[
            pltpu.VMEM((_NCHUNK + 1, _ROWS_CHUNK), jnp.int32),
            pltpu.VMEM((_NCHUNK + 1, _ROWS_CHUNK), jnp.float32),
            [pltpu.VMEM((_ROWS_CHUNK, 2 * _D), jnp.float32)
             for _ in range(_NBUF)],
            pltpu.VMEM((_PAIR * _FLUSH, _D), jnp.float32),
            [pltpu.SemaphoreType.DMA for _ in range(_NBUF)],
        ],
    )
    def k(x_hbm, emb_hbm, out_hbm, idx_v, par_v, bufs, stag, sems):
        lane_iota = lax.iota(jnp.int32, 16)
        wid = lax.axis_index("s") * _NC + lax.axis_index("c")
        pltpu.sync_copy(x_hbm.at[pl.ds(wid * _NCHUNK, _NCHUNK)],
                        idx_v.at[pl.ds(0, _NCHUNK)])

        @pl.loop(0, _NCHUNK)
        def _(j):
            rows = jnp.broadcast_to(j, (16,))
            for m in range(7):
                if m < 6:
                    xv = idx_v[j, pl.ds(m * 16, 16)]
                else:
                    cols = jnp.minimum(lane_iota + m * 16, _ROWS_CHUNK - 1)
                    xv = plsc.load_gather(idx_v, [rows, cols])
                blk = lax.div(xv, 2 * _CB)
                q = xv - blk * (2 * _CB)
                hi = lax.select(q >= _CB, jnp.ones((16,), jnp.int32),
                                jnp.zeros((16,), jnp.int32))
                pair = blk * _CB + q - hi * _CB
                par = lax.convert_element_type(hi, jnp.float32)
                if m < 6:
                    idx_v[j, pl.ds(m * 16, 16)] = pair
                    par_v[j, pl.ds(m * 16, 16)] = par
                else:
                    msk = lane_iota < (_ROWS_CHUNK - m * 16)
                    plsc.store_scatter(idx_v, [rows, cols], pair, mask=msk)
                    plsc.store_scatter(par_v, [rows, cols], par, mask=msk)

        for s in range(_NBUF):
            pltpu.async_copy(
                emb_hbm.at[idx_v.at[s].at[pl.ds(0, _ROWS_CHUNK)]],
                bufs[s], sems[s])

        def accum(j, buf):
            srow = _PAIR * lax.rem(j, _FLUSH)
            rows = jnp.broadcast_to(j, (16,))
            for h in range(_PAIR):
                acc = [jnp.zeros((16,), jnp.float32) for _ in range(_LG)]
                pv = None
                for r in range(_H):
                    rr = h * _H + r
                    if rr % 16 == 0 or pv is None:
                        blk16 = (rr // 16) * 16
                        if blk16 + 16 <= _ROWS_CHUNK:
                            pv = par_v[j, pl.ds(blk16, 16)]
                        else:
                            pcols = jnp.minimum(
                                lane_iota + blk16, _ROWS_CHUNK - 1)
                            pv = plsc.load_gather(par_v, [rows, pcols])
                    p = jnp.take(pv, lane_iota * 0 + (rr % 16))
                    for l in range(_LG):
                        e = buf[rr, pl.ds(l * 16, 16)]
                        o = buf[rr, pl.ds(_D + l * 16, 16)]
                        acc[l] = acc[l] + e + p * (o - e)
                for l in range(_LG):
                    stag[srow + h, pl.ds(l * 16, 16)] = acc[l]

        @pl.loop(0, _NCHUNK, step=_NBUF)
        def _(jbase):
            for s in range(_NBUF):
                j = jbase + s
                pltpu.make_async_copy(
                    emb_hbm.at[idx_v.at[j].at[pl.ds(0, _ROWS_CHUNK)]],
                    bufs[s], sems[s]).wait()
                accum(j, bufs[s])
                nxt = j + _NBUF

                @pl.when(nxt < _NCHUNK)
                def _():
                    pltpu.async_copy(
                        emb_hbm.at[idx_v.at[nxt].at[pl.ds(0, _ROWS_CHUNK)]],
                        bufs[s], sems[s])

            @pl.when(lax.rem(jbase, _FLUSH) == _FLUSH - _NBUF)
            def _():
                base = pl.multiple_of(
                    wid * _BAGS_W + _PAIR * (jbase - (_FLUSH - _NBUF)), 64)
                pltpu.sync_copy(
                    stag, out_hbm.at[pl.ds(base, _PAIR * _FLUSH)])

    return k(xr, emb2)


def _tc_head(bag, wp, bp):
    tb = 1024

    def body(bag_ref, w_ref, b_ref, out_ref):
        y = jnp.dot(bag_ref[...], w_ref[...],
                    preferred_element_type=jnp.float32) + b_ref[...]
        m = jnp.max(y, axis=1, keepdims=True)
        e = jnp.exp(y - m)
        out_ref[...] = e / jnp.sum(e, axis=1, keepdims=True)

    return pl.pallas_call(
        body,
        grid=(_B // tb,),
        in_specs=[
            pl.BlockSpec((tb, _D), lambda i: (i, 0)),
            pl.BlockSpec((_D, 128), lambda i: (0, 0)),
            pl.BlockSpec((1, 128), lambda i: (0, 0)),
        ],
        out_specs=pl.BlockSpec((tb, 128), lambda i: (i, 0)),
        out_shape=jax.ShapeDtypeStruct((_B, 128), jnp.float32),
    )(bag, wp, bp)


def kernel(x, emb, W, b):
    x = x.astype(jnp.int32)
    xr = x.reshape(_B * _H // _ROWS_CHUNK, _ROWS_CHUNK)
    emb2 = _tc_pairpack(emb.T)
    bag = _sc_bag_sum(xr, emb2)
    wp = jnp.zeros((_D, 128), jnp.float32).at[:, :_OUT].set(W.T * (1.0 / _H))
    bp = jnp.full((1, 128), -1e30, jnp.float32).at[0, :_OUT].set(b)
    out = _tc_head(bag, wp, bp)
    return out[:, :_OUT]


# speedup vs baseline: 1.6652x; 1.6652x over previous
import functools

import jax
import jax.numpy as jnp
from jax import lax
from jax.experimental import pallas as pl
from jax.experimental.pallas import tpu as pltpu
from jax.experimental.pallas import tpu_sc as plsc

_VOCAB = 1000000
_D = 64
_OUT = 100
_B = 16384
_H = 50

_NC, _NS = 2, 16
_NW = _NC * _NS
_BAGS_W = _B // _NW
_PAIR = 2
_ROWS_CHUNK = _PAIR * _H
_NCHUNK = _BAGS_W // _PAIR
_NBUF = 4
_LG = _D // 16
_FLUSH = 32


def _tc_pairpack(embT):
    cb = 16384
    grid = (_VOCAB + cb - 1) // cb

    def body(in_ref, eye_ref, out_ref):
        out_ref[...] = jax.lax.dot_general(
            in_ref[...], eye_ref[...],
            dimension_numbers=(((0,), (0,)), ((), ())),
            preferred_element_type=jnp.float32)

    return pl.pallas_call(
        body,
        grid=(grid,),
        in_specs=[
            pl.BlockSpec((_D, cb), lambda i: (0, i)),
            pl.BlockSpec((_D, 2 * _D), lambda i: (0, 0)),
        ],
        out_specs=pl.BlockSpec((cb, 2 * _D), lambda i: (i, 0)),
        out_shape=jax.ShapeDtypeStruct((_VOCAB, 2 * _D), jnp.float32),
    )(embT, jnp.concatenate(
        [jnp.eye(_D, dtype=jnp.float32)] * 2, axis=1))


def _sc_bag_sum(xr, emb2):
    mesh = plsc.VectorSubcoreMesh(
        core_axis_name="c", subcore_axis_name="s",
        num_cores=_NC, num_subcores=_NS)

    @functools.partial(
        pl.kernel,
        out_type=jax.ShapeDtypeStruct((_B, _D), jnp.float32),
        mesh=mesh,
        compiler_params=pltpu.CompilerParams(
            use_tc_tiling_on_sc=True, needs_layout_passes=False),
        scratch_types=[
            pltpu.VMEM((_NCHUNK, _ROWS_CHUNK), jnp.int32),
            [pltpu.VMEM((_ROWS_CHUNK, 2 * _D), jnp.float32)
             for _ in range(_NBUF)],
            pltpu.VMEM((_PAIR * _FLUSH, _D), jnp.float32),
            [pltpu.SemaphoreType.DMA for _ in range(_NBUF)],
        ],
    )
    def k(x_hbm, emb_hbm, out_hbm, idx_v, bufs, stag, sems):
        wid = lax.axis_index("s") * _NC + lax.axis_index("c")
        pltpu.sync_copy(x_hbm.at[pl.ds(wid * _NCHUNK, _NCHUNK)], idx_v)

        for s in range(_NBUF):
            pltpu.async_copy(
                emb_hbm.at[idx_v.at[s].at[pl.ds(0, _ROWS_CHUNK)]],
                bufs[s], sems[s])

        def accum(j, buf):
            srow = _PAIR * lax.rem(j, _FLUSH)
            for h in range(_PAIR):
                r0 = h * _H
                acc = [buf[r0, pl.ds(l * 16, 16)] for l in range(_LG)]
                for r in range(1, _H):
                    for l in range(_LG):
                        acc[l] = acc[l] + buf[r0 + r, pl.ds(l * 16, 16)]
                for l in range(_LG):
                    stag[srow + h, pl.ds(l * 16, 16)] = acc[l]

        @pl.loop(0, _NCHUNK, step=_NBUF)
        def _(jbase):
            for s in range(_NBUF):
                j = jbase + s
                pltpu.make_async_copy(
                    emb_hbm.at[idx_v.at[j].at[pl.ds(0, _ROWS_CHUNK)]],
                    bufs[s], sems[s]).wait()
                accum(j, bufs[s])
                nxt = j + _NBUF

                @pl.when(nxt < _NCHUNK)
                def _():
                    pltpu.async_copy(
                        emb_hbm.at[idx_v.at[nxt].at[pl.ds(0, _ROWS_CHUNK)]],
                        bufs[s], sems[s])

            @pl.when(lax.rem(jbase, _FLUSH) == _FLUSH - _NBUF)
            def _():
                base = pl.multiple_of(
                    wid * _BAGS_W + _PAIR * (jbase - (_FLUSH - _NBUF)), 64)
                pltpu.sync_copy(
                    stag, out_hbm.at[pl.ds(base, _PAIR * _FLUSH)])

    return k(xr, emb2)


def _tc_head(bag, wp, bp):
    tb = 1024

    def body(bag_ref, w_ref, b_ref, out_ref):
        y = jnp.dot(bag_ref[...], w_ref[...],
                    preferred_element_type=jnp.float32) + b_ref[...]
        m = jnp.max(y, axis=1, keepdims=True)
        e = jnp.exp(y - m)
        out_ref[...] = e / jnp.sum(e, axis=1, keepdims=True)

    return pl.pallas_call(
        body,
        grid=(_B // tb,),
        in_specs=[
            pl.BlockSpec((tb, _D), lambda i: (i, 0)),
            pl.BlockSpec((_D, 128), lambda i: (0, 0)),
            pl.BlockSpec((1, 128), lambda i: (0, 0)),
        ],
        out_specs=pl.BlockSpec((tb, 128), lambda i: (i, 0)),
        out_shape=jax.ShapeDtypeStruct((_B, 128), jnp.float32),
    )(bag, wp, bp)


def kernel(x, emb, W, b):
    x = x.astype(jnp.int32)
    xr = x.reshape(_B * _H // _ROWS_CHUNK, _ROWS_CHUNK)
    emb2 = _tc_pairpack(emb.T)
    bag = _sc_bag_sum(xr, emb2)
    wp = jnp.zeros((_D, 128), jnp.float32).at[:, :_OUT].set(W.T * (1.0 / _H))
    bp = jnp.full((1, 128), -1e30, jnp.float32).at[0, :_OUT].set(b)
    out = _tc_head(bag, wp, bp)
    return out[:, :_OUT]
